# baseline (speedup 1.0000x reference)
import functools

import jax
import jax.numpy as jnp
from jax import lax
from jax.experimental import pallas as pl
from jax.experimental.pallas import tpu as pltpu
from jax.experimental.pallas import tpu_sc as plsc

_LANES = 16


@functools.cache
def _make_gather(V, D, B):
    info = plsc.get_sparse_core_info()
    NC, NS = info.num_cores, info.num_subcores
    NW = NC * NS
    assert B % (_LANES * NW) == 0 and B % (8 * NW) == 0
    b_per_w = B // NW
    half = b_per_w // 2
    mesh = plsc.VectorSubcoreMesh(core_axis_name="c", subcore_axis_name="s")

    @functools.partial(
        pl.kernel,
        mesh=mesh,
        out_type=jax.ShapeDtypeStruct((B // 8, 8, D), jnp.float32),
        scratch_types=[
            pltpu.VMEM((b_per_w,), jnp.int32),
            pltpu.VMEM((half // 8, 8, D), jnp.float32),
            pltpu.VMEM_SHARED((NS, half // 8, 8, D), jnp.float32),
            pltpu.SemaphoreType.DMA,
            pltpu.SemaphoreType.DMA,
        ],
        compiler_params=pltpu.CompilerParams(needs_layout_passes=False),
    )
    def gather_kernel(idx_hbm, table_hbm, out_hbm, g_v, rows_v, sh_v, sem0, sem1):
        sid = lax.axis_index("s")
        wid = sid * NC + lax.axis_index("c")
        base = wid * b_per_w
        pltpu.sync_copy(idx_hbm.at[pl.ds(base, b_per_w)], g_v)
        my_sh = sh_v.at[sid]

        def fire(j, _):
            g16 = g_v[pl.ds(j * _LANES, _LANES)]
            for l in range(8):
                pltpu.async_copy(table_hbm.at[g16[l]], rows_v.at[j, l], sem0)
                pltpu.async_copy(table_hbm.at[g16[l + 8]], my_sh.at[j, l], sem1)
            return 0

        lax.fori_loop(0, b_per_w // _LANES, fire, 0)

        def drain(i, _):
            pltpu.make_async_copy(table_hbm.at[0], rows_v.at[0, 0], sem0).wait()
            pltpu.make_async_copy(table_hbm.at[0], my_sh.at[0, 0], sem1).wait()
            return 0

        lax.fori_loop(0, half, drain, 0)
        out_w = out_hbm.at[pl.ds(wid * (b_per_w // 8), b_per_w // 8)]

        def wb(k, _):
            pltpu.async_copy(rows_v.at[k], out_hbm.at[wid * (b_per_w // 8) + 2 * k], sem0)
            pltpu.async_copy(my_sh.at[k], out_hbm.at[wid * (b_per_w // 8) + 2 * k + 1], sem1)
            return 0

        del out_w
        lax.fori_loop(0, half // 8, wb, 0)

        def wb_drain(k, _):
            pltpu.make_async_copy(rows_v.at[0], out_hbm.at[0], sem0).wait()
            pltpu.make_async_copy(my_sh.at[0], out_hbm.at[0], sem1).wait()
            return 0

        lax.fori_loop(0, half // 8, wb_drain, 0)

    return gather_kernel


@jax.jit
def kernel(g, table):
    V, D = table.shape
    B = g.shape[0]
    f = _make_gather(V, D, B)
    g32 = g.astype(jnp.int32)
    return f(g32, table).reshape(B, D)
